# baseline (speedup 1.0000x reference)
import functools
import math

import jax
import jax.numpy as jnp
from jax import lax
from jax.experimental import pallas as pl
from jax.experimental.pallas import tpu as pltpu
from jax.experimental.pallas import tpu_sc as plsc

_NC = 2
_NS = 16
_NW = _NC * _NS

_NBUF = 4
_U = 128


def _emb_kernel(units_per_w, n_jc, d, scale,
                idx_hbm, table_hbm, out_hbm, idx_all, bufs, tbufs, gsem, wsem):
    wid = lax.axis_index("s") * _NC + lax.axis_index("c")
    base_u = wid * units_per_w
    pltpu.sync_copy(idx_hbm.at[pl.ds(base_u * _U, units_per_w * _U)], idx_all)

    iota16 = lax.iota(jnp.int32, 16)

    def gather_start(uu, b):
        pltpu.async_copy(
            table_hbm.at[idx_all.at[pl.ds(uu * _U, _U)]], bufs[b], gsem[b])

    def gather_wait(b):
        pltpu.make_async_copy(
            table_hbm.at[idx_all.at[pl.ds(0, _U)]], bufs[b], gsem[b]).wait()

    def wb_start(uu, b):
        u = base_u + uu
        j = u >> 7
        c = u & 127
        for r in range(4):
            pltpu.async_copy(tbufs[b].at[pl.ds(r * 8, 8)],
                             out_hbm.at[j, r, c], wsem[b])

    def wb_wait(b):
        for r in range(4):
            pltpu.make_async_copy(tbufs[b].at[pl.ds(0, 8)],
                                  out_hbm.at[0, 0, 0], wsem[b]).wait()

    row_ids = [l0 + iota16 for l0 in range(0, _U, 16)]

    def transpose_scale(b):
        buf, tbuf = bufs[b], tbufs[b]

        @plsc.parallel_loop(0, d, 1, unroll=4)
        def _(e):
            col = jnp.full((16,), e, jnp.int32)
            for k, rid in enumerate(row_ids):
                v = plsc.load_gather(buf, [rid, col])
                tbuf[e, pl.ds(k * 16, 16)] = v * scale

    for b in range(_NBUF - 1):
        gather_start(b, b)

    def group(q, _):
        for b in range(_NBUF):
            uu = q * _NBUF + b
            gather_wait(b)
            transpose_scale(b)
            wb_start(uu, b)
            u2 = uu + _NBUF - 1
            pb = (b - 1) % _NBUF

            @pl.when(u2 < units_per_w)
            def _():
                if b == 0:
                    @pl.when(uu > 0)
                    def _():
                        wb_wait(pb)
                else:
                    wb_wait(pb)
                gather_start(u2, pb)
        return 0

    lax.fori_loop(0, units_per_w // _NBUF, group, 0)

    for b in range(_NBUF):
        wb_wait(b)


def _repack_tc_kernel(x_ref, s_ref, e_ref, o_ref):
    x = x_ref[...]
    acc = jnp.zeros(o_ref.shape, jnp.float32)
    for k in range(4):
        acc = acc + jnp.dot(
            jnp.dot(s_ref[k], x, preferred_element_type=jnp.float32),
            e_ref[k], preferred_element_type=jnp.float32)
    o_ref[...] = acc


def _depad_kernel(tok_t, out3, buf8, buf2, sem):
    wid = lax.axis_index("s") * _NC + lax.axis_index("c")
    g = wid // 2
    par = wid % 2
    a0 = par * 3

    def blk(i, _):
        a = a0 + i
        for cc in range(8):
            pltpu.async_copy(
                tok_t.at[pl.ds(a * 8, 8), pl.ds((g * 8 + cc) * 128, 128)],
                buf8.at[:, cc], sem)
        for cc in range(8):
            pltpu.make_async_copy(
                tok_t.at[pl.ds(0, 8), pl.ds(0, 128)], buf8.at[:, cc],
                sem).wait()
        pltpu.sync_copy(buf8, out3.at[pl.ds(a * 8, 8), pl.ds(g * 8, 8)])
        return 0

    lax.fori_loop(0, 3, blk, 0)

    @pl.when(par == 0)
    def _():
        for cc in range(8):
            pltpu.async_copy(
                tok_t.at[pl.ds(48, 2), pl.ds((g * 8 + cc) * 128, 128)],
                buf2.at[:, cc], sem)
        for cc in range(8):
            pltpu.make_async_copy(
                tok_t.at[pl.ds(0, 2), pl.ds(0, 128)], buf2.at[:, cc],
                sem).wait()
        pltpu.sync_copy(buf2, out3.at[pl.ds(48, 2), pl.ds(g * 8, 8)])


def kernel(tokens, table):
    s, t = tokens.shape
    v, d = table.shape
    b = s * t
    n_jc = (s // _U) * t
    assert s % _U == 0 and n_jc % (_NW * _NBUF) == 0 and d == 32
    units_per_w = n_jc // _NW
    scale = float(math.sqrt(d))

    mesh = plsc.VectorSubcoreMesh(core_axis_name="c", subcore_axis_name="s")

    depad = pl.kernel(
        _depad_kernel,
        mesh=mesh,
        out_type=jax.ShapeDtypeStruct((t, s // _U, _U), jnp.int32),
        scratch_types=[
            pltpu.VMEM((8, 8, _U), jnp.int32),
            pltpu.VMEM((2, 8, _U), jnp.int32),
            pltpu.SemaphoreType.DMA,
        ],
        compiler_params=pltpu.CompilerParams(use_tc_tiling_on_sc=True,
                                             needs_layout_passes=False),
    )
    idx = depad(tokens.T).reshape(b)

    rblk = 1024
    rgrid = -(-v // rblk)
    sel = (jnp.arange(rblk, dtype=jnp.int32)[None, None, :] ==
           (4 * jnp.arange(rblk // 4, dtype=jnp.int32)[None, :, None] +
            jnp.arange(4, dtype=jnp.int32)[:, None, None])
           ).astype(jnp.float32)
    emb = (jnp.arange(128, dtype=jnp.int32)[None, None, :] ==
           (32 * jnp.arange(4, dtype=jnp.int32)[:, None, None] +
            jnp.arange(d, dtype=jnp.int32)[None, :, None])
           ).astype(jnp.float32)
    tbl_rm = pl.pallas_call(
        _repack_tc_kernel,
        grid=(rgrid,),
        in_specs=[
            pl.BlockSpec((rblk, d), lambda g: (g, 0)),
            pl.BlockSpec((4, rblk // 4, rblk), lambda g: (0, 0, 0)),
            pl.BlockSpec((4, d, 128), lambda g: (0, 0, 0)),
        ],
        out_specs=pl.BlockSpec((rblk * d // 128, 128), lambda g: (g, 0)),
        out_shape=jax.ShapeDtypeStruct((v * d // 128, 128), jnp.float32),
    )(table, sel, emb)
    table_rm = tbl_rm.reshape(v, d)
    run = pl.kernel(
        functools.partial(_emb_kernel, units_per_w, n_jc, d, scale),
        mesh=mesh,
        out_type=jax.ShapeDtypeStruct((t, d // 8, s // _U, 8, _U),
                                      jnp.float32),
        scratch_types=[
            pltpu.VMEM((units_per_w * _U,), jnp.int32),
            [pltpu.VMEM((_U, d), jnp.float32) for _ in range(_NBUF)],
            [pltpu.VMEM((d, _U), jnp.float32) for _ in range(_NBUF)],
            [pltpu.SemaphoreType.DMA for _ in range(_NBUF)],
            [pltpu.SemaphoreType.DMA for _ in range(_NBUF)],
        ],
        compiler_params=pltpu.CompilerParams(use_tc_tiling_on_sc=False,
                                             needs_layout_passes=False),
    )
    out5 = run(idx, table_rm)
    return out5.transpose(2, 4, 0, 1, 3).reshape(s, t, d)
